# baseline (speedup 1.0000x reference)
import functools

import jax
import jax.numpy as jnp
from jax import lax
from jax.experimental import pallas as pl
from jax.experimental.pallas import tpu as pltpu
from jax.experimental.pallas import tpu_sc as plsc


def _tc_body(l_ref, x_ref, c_ref, fm_ref, a_out, ats_out, gl_out, fl_out,
             m_acc, fl_acc):
    i = pl.program_id(0)
    nblk = pl.num_programs(0)
    a = 1.0 / (1.0 + jnp.exp(-l_ref[...]))
    ats = a * c_ref[...]
    a_out[...] = a.astype(jnp.bfloat16)
    ats_out[...] = ats.astype(jnp.bfloat16)
    m = lax.dot_general(a, ats, (((0,), (0,)), ((), ())),
                        preferred_element_type=jnp.float32)
    recon = jnp.dot(ats, fm_ref[...], preferred_element_type=jnp.float32)
    d = x_ref[...] - recon
    fl = jnp.sum(d * d)

    @pl.when(i == 0)
    def _():
        m_acc[...] = m
        fl_acc[0, 0] = fl

    @pl.when(i > 0)
    def _():
        m_acc[...] += m
        fl_acc[0, 0] += fl

    @pl.when(i == nblk - 1)
    def _():
        mm = m_acc[...]
        gl_out[0, 0] = jnp.sum(mm * mm.T)
        fl_out[0, 0] = fl_acc[0, 0]


def _tc_call(logits, x, c2, feat_mat, blk):
    n, k = logits.shape
    c = x.shape[1]
    grid = (n // blk,)
    return pl.pallas_call(
        _tc_body,
        grid=grid,
        in_specs=[
            pl.BlockSpec((blk, k), lambda i: (i, 0)),
            pl.BlockSpec((blk, c), lambda i: (i, 0)),
            pl.BlockSpec((1, k), lambda i: (0, 0)),
            pl.BlockSpec((k, c), lambda i: (0, 0)),
        ],
        out_specs=[
            pl.BlockSpec((blk, k), lambda i: (i, 0)),
            pl.BlockSpec((blk, k), lambda i: (i, 0)),
            pl.BlockSpec((1, 1), lambda i: (0, 0), memory_space=pltpu.SMEM),
            pl.BlockSpec((1, 1), lambda i: (0, 0), memory_space=pltpu.SMEM),
        ],
        out_shape=[
            jax.ShapeDtypeStruct((n, k), jnp.bfloat16),
            jax.ShapeDtypeStruct((n, k), jnp.bfloat16),
            jax.ShapeDtypeStruct((1, 1), jnp.float32),
            jax.ShapeDtypeStruct((1, 1), jnp.float32),
        ],
        scratch_shapes=[
            pltpu.VMEM((k, k), jnp.float32),
            pltpu.SMEM((1, 1), jnp.float32),
        ],
    )(logits, x, c2, feat_mat)


_G = 256
_NW = 32
_NBUF = 5
_L = 16
_UNROLL = 4
_SUB = 32


def _sc_edge_kernel(t_chunks, k, n_pad):
    mesh = plsc.VectorSubcoreMesh(core_axis_name="c", subcore_axis_name="s")
    kh = k // 32

    @functools.partial(
        pl.kernel,
        out_type=jax.ShapeDtypeStruct((_NW, 2 * k), jnp.float32),
        mesh=mesh,
        compiler_params=pltpu.CompilerParams(
            use_tc_tiling_on_sc=False, needs_layout_passes=False),
        scratch_types=[
            pltpu.VMEM((t_chunks, _G), jnp.int32),
            pltpu.VMEM((t_chunks, _G), jnp.int32),
            pltpu.VMEM((_NBUF, _G, k), jnp.bfloat16),
            pltpu.VMEM((_NBUF, _G, k), jnp.bfloat16),
            pltpu.VMEM((2 * k,), jnp.float32),
        ] + [pltpu.SemaphoreType.DMA] * (2 * _NBUF),
    )
    def edge_sum(src_hbm, dst_hbm, ats_hbm, a_hbm, out_hbm,
                 si, di, ra, rb, accv, *sems):
        wid = lax.axis_index("s") * 2 + lax.axis_index("c")

        pltpu.sync_copy(src_hbm.at[wid], si)
        pltpu.sync_copy(dst_hbm.at[wid], di)

        for j in range(2 * kh):
            accv[pl.ds(j * _L, _L)] = jnp.zeros((_L,), jnp.float32)

        def start(t, b):
            pltpu.make_async_copy(ats_hbm.at[di.at[t]], ra.at[b], sems[b]).start()
            pltpu.make_async_copy(a_hbm.at[si.at[t]], rb.at[b],
                                  sems[_NBUF + b]).start()

        def wait(t, b):
            pltpu.make_async_copy(ats_hbm.at[di.at[t]], ra.at[b], sems[b]).wait()
            pltpu.make_async_copy(a_hbm.at[si.at[t]], rb.at[b],
                                  sems[_NBUF + b]).wait()

        for t0 in range(min(_NBUF - 1, t_chunks)):
            start(t0, t0)

        def chunk_body(t, _):
            b = lax.rem(t, _NBUF)

            @pl.when(t + _NBUF - 1 < t_chunks)
            def _():
                bn = lax.rem(t + _NBUF - 1, _NBUF)
                for bb in range(_NBUF):
                    @pl.when(bn == bb)
                    def _():
                        start(t + _NBUF - 1, bb)

            for bb in range(_NBUF):
                @pl.when(b == bb)
                def _():
                    wait(t, bb)

                    def row_body(i, carry):
                        cur = list(carry)
                        for r in range(_UNROLL):
                            row = i * _UNROLL + r
                            for j in range(kh):
                                p = (ra[bb, row, pl.ds(j * 32, 32)]
                                     * rb[bb, row, pl.ds(j * 32, 32)])
                                cur[j] = cur[j] + p
                        return tuple(cur)

                    z = jnp.zeros((32,), jnp.bfloat16)
                    nsub = _G // _SUB
                    for q in range(nsub):
                        lo_it = q * (_SUB // _UNROLL)
                        hi_it = (q + 1) * (_SUB // _UNROLL)
                        acc = lax.fori_loop(lo_it, hi_it, row_body, (z,) * kh)
                        for j in range(kh):
                            lo, hi = plsc.unpack(
                                acc[j], format=plsc.PackFormat.INTERLEAVED)
                            o0 = (2 * j) * _L
                            o1 = (2 * j + 1) * _L
                            accv[pl.ds(o0, _L)] = accv[pl.ds(o0, _L)] + lo
                            accv[pl.ds(o1, _L)] = accv[pl.ds(o1, _L)] + hi
            return 0

        lax.fori_loop(0, t_chunks, chunk_body, 0)
        pltpu.sync_copy(accv, out_hbm.at[wid])

    return edge_sum


def kernel(x, edge_index, affiliate_logits, community_scalars, feat_mat):
    n, c = x.shape
    k = affiliate_logits.shape[1]
    e = edge_index.shape[1]

    a_bf, ats_bf, gl, fl = _tc_call(
        affiliate_logits, x, community_scalars.reshape(1, k), feat_mat,
        blk=2000)

    n_pad = n + 16
    a_pad = jnp.pad(a_bf, ((0, n_pad - n), (0, 0)))
    ats_pad = jnp.pad(ats_bf, ((0, n_pad - n), (0, 0)))
    nch = -(-e // _G)
    t_chunks = -(-nch // _NW)
    e_pad = t_chunks * _NW * _G
    src = jnp.concatenate(
        [edge_index[0], jnp.full((e_pad - e,), n, jnp.int32)])
    dst = jnp.concatenate(
        [edge_index[1], jnp.full((e_pad - e,), n, jnp.int32)])
    src3 = src.reshape(t_chunks, _NW, _G).transpose(1, 0, 2)
    dst3 = dst.reshape(t_chunks, _NW, _G).transpose(1, 0, 2)

    partials = _sc_edge_kernel(t_chunks, k, n_pad)(src3, dst3, ats_pad, a_pad)
    local = 2.0 * jnp.sum(partials)

    loss = (gl[0, 0] - local + jnp.float32(e)) / jnp.float32(n)
    return loss + fl[0, 0] / jnp.float32(c)


# speedup vs baseline: 1.2256x; 1.2256x over previous
import functools

import jax
import jax.numpy as jnp
from jax import lax
from jax.experimental import pallas as pl
from jax.experimental.pallas import tpu as pltpu
from jax.experimental.pallas import tpu_sc as plsc


def _tc_body(l_ref, x_ref, c_ref, fm_ref, a_out, ats_out, gl_out, fl_out,
             m_acc, fl_acc):
    i = pl.program_id(0)
    nblk = pl.num_programs(0)
    a = 1.0 / (1.0 + jnp.exp(-l_ref[...]))
    ats = a * c_ref[...]
    a_out[...] = a.astype(jnp.bfloat16)
    ats_out[...] = ats.astype(jnp.bfloat16)
    m = lax.dot_general(a, ats, (((0,), (0,)), ((), ())),
                        preferred_element_type=jnp.float32)
    recon = jnp.dot(ats, fm_ref[...], preferred_element_type=jnp.float32)
    d = x_ref[...] - recon
    fl = jnp.sum(d * d)

    @pl.when(i == 0)
    def _():
        m_acc[...] = m
        fl_acc[0, 0] = fl

    @pl.when(i > 0)
    def _():
        m_acc[...] += m
        fl_acc[0, 0] += fl

    @pl.when(i == nblk - 1)
    def _():
        mm = m_acc[...]
        gl_out[0, 0] = jnp.sum(mm * mm.T)
        fl_out[0, 0] = fl_acc[0, 0]


def _tc_call(logits, x, c2, feat_mat, blk):
    n, k = logits.shape
    c = x.shape[1]
    grid = (n // blk,)
    return pl.pallas_call(
        _tc_body,
        grid=grid,
        in_specs=[
            pl.BlockSpec((blk, k), lambda i: (i, 0)),
            pl.BlockSpec((blk, c), lambda i: (i, 0)),
            pl.BlockSpec((1, k), lambda i: (0, 0)),
            pl.BlockSpec((k, c), lambda i: (0, 0)),
        ],
        out_specs=[
            pl.BlockSpec((blk, k), lambda i: (i, 0)),
            pl.BlockSpec((blk, k), lambda i: (i, 0)),
            pl.BlockSpec((1, 1), lambda i: (0, 0), memory_space=pltpu.SMEM),
            pl.BlockSpec((1, 1), lambda i: (0, 0), memory_space=pltpu.SMEM),
        ],
        out_shape=[
            jax.ShapeDtypeStruct((n, k), jnp.bfloat16),
            jax.ShapeDtypeStruct((n, k), jnp.bfloat16),
            jax.ShapeDtypeStruct((1, 1), jnp.float32),
            jax.ShapeDtypeStruct((1, 1), jnp.float32),
        ],
        scratch_shapes=[
            pltpu.VMEM((k, k), jnp.float32),
            pltpu.SMEM((1, 1), jnp.float32),
        ],
    )(logits, x, c2, feat_mat)


_G = 128
_NW = 32
_NBUF = 10
_L = 16
_UNROLL = 4
_SUB = 32


def _sc_edge_kernel(t_chunks, k, n_pad):
    mesh = plsc.VectorSubcoreMesh(core_axis_name="c", subcore_axis_name="s")
    kh = k // 32

    @functools.partial(
        pl.kernel,
        out_type=jax.ShapeDtypeStruct((_NW, 2 * k), jnp.float32),
        mesh=mesh,
        compiler_params=pltpu.CompilerParams(
            use_tc_tiling_on_sc=False, needs_layout_passes=False),
        scratch_types=[
            pltpu.VMEM((t_chunks, _G), jnp.int32),
            pltpu.VMEM((t_chunks, _G), jnp.int32),
            pltpu.VMEM((_NBUF, _G, k), jnp.bfloat16),
            pltpu.VMEM((_NBUF, _G, k), jnp.bfloat16),
            pltpu.VMEM((2 * k,), jnp.float32),
        ] + [pltpu.SemaphoreType.DMA] * (2 * _NBUF),
    )
    def edge_sum(src_hbm, dst_hbm, ats_hbm, a_hbm, out_hbm,
                 si, di, ra, rb, accv, *sems):
        wid = lax.axis_index("s") * 2 + lax.axis_index("c")

        pltpu.sync_copy(src_hbm.at[wid], si)
        pltpu.sync_copy(dst_hbm.at[wid], di)

        for j in range(2 * kh):
            accv[pl.ds(j * _L, _L)] = jnp.zeros((_L,), jnp.float32)

        def start(t, b):
            pltpu.make_async_copy(ats_hbm.at[di.at[t]], ra.at[b], sems[b]).start()
            pltpu.make_async_copy(a_hbm.at[si.at[t]], rb.at[b],
                                  sems[_NBUF + b]).start()

        def wait(t, b):
            pltpu.make_async_copy(ats_hbm.at[di.at[t]], ra.at[b], sems[b]).wait()
            pltpu.make_async_copy(a_hbm.at[si.at[t]], rb.at[b],
                                  sems[_NBUF + b]).wait()

        for t0 in range(min(_NBUF - 1, t_chunks)):
            start(t0, t0)

        def chunk_body(t, _):
            b = lax.rem(t, _NBUF)

            @pl.when(t + _NBUF - 1 < t_chunks)
            def _():
                bn = lax.rem(t + _NBUF - 1, _NBUF)
                for bb in range(_NBUF):
                    @pl.when(bn == bb)
                    def _():
                        start(t + _NBUF - 1, bb)

            for bb in range(_NBUF):
                @pl.when(b == bb)
                def _():
                    wait(t, bb)

                    def row_body(i, carry):
                        cur = list(carry)
                        for r in range(_UNROLL):
                            row = i * _UNROLL + r
                            for j in range(kh):
                                p = (ra[bb, row, pl.ds(j * 32, 32)]
                                     * rb[bb, row, pl.ds(j * 32, 32)])
                                cur[j] = cur[j] + p
                        return tuple(cur)

                    z = jnp.zeros((32,), jnp.bfloat16)
                    nsub = _G // _SUB
                    for q in range(nsub):
                        lo_it = q * (_SUB // _UNROLL)
                        hi_it = (q + 1) * (_SUB // _UNROLL)
                        acc = lax.fori_loop(lo_it, hi_it, row_body, (z,) * kh)
                        for j in range(kh):
                            lo, hi = plsc.unpack(
                                acc[j], format=plsc.PackFormat.INTERLEAVED)
                            o0 = (2 * j) * _L
                            o1 = (2 * j + 1) * _L
                            accv[pl.ds(o0, _L)] = accv[pl.ds(o0, _L)] + lo
                            accv[pl.ds(o1, _L)] = accv[pl.ds(o1, _L)] + hi
            return 0

        lax.fori_loop(0, t_chunks, chunk_body, 0)
        pltpu.sync_copy(accv, out_hbm.at[wid])

    return edge_sum


def kernel(x, edge_index, affiliate_logits, community_scalars, feat_mat):
    n, c = x.shape
    k = affiliate_logits.shape[1]
    e = edge_index.shape[1]

    a_bf, ats_bf, gl, fl = _tc_call(
        affiliate_logits, x, community_scalars.reshape(1, k), feat_mat,
        blk=2000)

    n_pad = n + 16
    a_pad = jnp.pad(a_bf, ((0, n_pad - n), (0, 0)))
    ats_pad = jnp.pad(ats_bf, ((0, n_pad - n), (0, 0)))
    nch = -(-e // _G)
    t_chunks = -(-nch // _NW)
    e_pad = t_chunks * _NW * _G
    src = jnp.concatenate(
        [edge_index[0], jnp.full((e_pad - e,), n, jnp.int32)])
    dst = jnp.concatenate(
        [edge_index[1], jnp.full((e_pad - e,), n, jnp.int32)])
    src3 = src.reshape(t_chunks, _NW, _G).transpose(1, 0, 2)
    dst3 = dst.reshape(t_chunks, _NW, _G).transpose(1, 0, 2)

    partials = _sc_edge_kernel(t_chunks, k, n_pad)(src3, dst3, ats_pad, a_pad)
    local = 2.0 * jnp.sum(partials)

    loss = (gl[0, 0] - local + jnp.float32(e)) / jnp.float32(n)
    return loss + fl[0, 0] / jnp.float32(c)


# speedup vs baseline: 1.3202x; 1.0771x over previous
import functools

import jax
import jax.numpy as jnp
from jax import lax
from jax.experimental import pallas as pl
from jax.experimental.pallas import tpu as pltpu
from jax.experimental.pallas import tpu_sc as plsc


def _tc_body(l_ref, x_ref, c_ref, fm_ref, a_out, ats_out, gl_out, fl_out,
             m_acc, fl_acc):
    i = pl.program_id(0)
    nblk = pl.num_programs(0)
    a = 1.0 / (1.0 + jnp.exp(-l_ref[...]))
    ats = a * c_ref[...]
    a_out[...] = a
    ats_out[...] = ats
    m = lax.dot_general(a, ats, (((0,), (0,)), ((), ())),
                        preferred_element_type=jnp.float32)
    recon = jnp.dot(ats, fm_ref[...], preferred_element_type=jnp.float32)
    d = x_ref[...] - recon
    fl = jnp.sum(d * d)

    @pl.when(i == 0)
    def _():
        m_acc[...] = m
        fl_acc[0, 0] = fl

    @pl.when(i > 0)
    def _():
        m_acc[...] += m
        fl_acc[0, 0] += fl

    @pl.when(i == nblk - 1)
    def _():
        mm = m_acc[...]
        gl_out[0, 0] = jnp.sum(mm * mm.T)
        fl_out[0, 0] = fl_acc[0, 0]


def _tc_call(logits, x, c2, feat_mat, blk):
    n, k = logits.shape
    c = x.shape[1]
    grid = (n // blk,)
    return pl.pallas_call(
        _tc_body,
        grid=grid,
        in_specs=[
            pl.BlockSpec((blk, k), lambda i: (i, 0)),
            pl.BlockSpec((blk, c), lambda i: (i, 0)),
            pl.BlockSpec((1, k), lambda i: (0, 0)),
            pl.BlockSpec((k, c), lambda i: (0, 0)),
        ],
        out_specs=[
            pl.BlockSpec((blk, k), lambda i: (i, 0)),
            pl.BlockSpec((blk, k), lambda i: (i, 0)),
            pl.BlockSpec((1, 1), lambda i: (0, 0), memory_space=pltpu.SMEM),
            pl.BlockSpec((1, 1), lambda i: (0, 0), memory_space=pltpu.SMEM),
        ],
        out_shape=[
            jax.ShapeDtypeStruct((n, k), jnp.float32),
            jax.ShapeDtypeStruct((n, k), jnp.float32),
            jax.ShapeDtypeStruct((1, 1), jnp.float32),
            jax.ShapeDtypeStruct((1, 1), jnp.float32),
        ],
        scratch_shapes=[
            pltpu.VMEM((k, k), jnp.float32),
            pltpu.SMEM((1, 1), jnp.float32),
        ],
    )(logits, x, c2, feat_mat)


_GC = 2000
_NW = 32
_NB = 3
_L = 16
_UNR = 5


def _sc_edge_kernel(n, k, e):
    mesh = plsc.VectorSubcoreMesh(core_axis_name="c", subcore_axis_name="s")
    nchk = e // _GC
    groups = _GC // _L

    @functools.partial(
        pl.kernel,
        out_type=jax.ShapeDtypeStruct((_NW, 2 * _L), jnp.float32),
        mesh=mesh,
        compiler_params=pltpu.CompilerParams(
            use_tc_tiling_on_sc=False, needs_layout_passes=False),
        scratch_types=[
            pltpu.VMEM((_NB, _GC), jnp.int32),
            pltpu.VMEM((_NB, _GC), jnp.int32),
            pltpu.VMEM((2 * n,), jnp.float32),
            pltpu.VMEM((2 * n,), jnp.float32),
            pltpu.VMEM((2 * _L,), jnp.float32),
        ] + [pltpu.SemaphoreType.DMA] * _NB,
    )
    def edge_sum(src_hbm, dst_hbm, ats_hbm, a_hbm, out_hbm,
                 sb, db, tats, ta, accv, *sems):
        tid = lax.axis_index("s") * 2 + lax.axis_index("c")

        pltpu.sync_copy(ats_hbm.at[tid], tats)
        pltpu.sync_copy(a_hbm.at[tid], ta)

        accv[pl.ds(0, _L)] = jnp.zeros((_L,), jnp.float32)
        accv[pl.ds(_L, _L)] = jnp.zeros((_L,), jnp.float32)

        def start(t, b):
            base = t * _GC
            pltpu.make_async_copy(
                src_hbm.at[pl.ds(base, _GC)], sb.at[b], sems[b]).start()
            pltpu.make_async_copy(
                dst_hbm.at[pl.ds(base, _GC)], db.at[b], sems[b]).start()

        def wait(t, b):
            base = t * _GC
            pltpu.make_async_copy(
                src_hbm.at[pl.ds(base, _GC)], sb.at[b], sems[b]).wait()
            pltpu.make_async_copy(
                dst_hbm.at[pl.ds(base, _GC)], db.at[b], sems[b]).wait()

        for t0 in range(_NB - 1):
            start(t0, t0)

        nv = jnp.full((_L,), n, jnp.int32)

        def chunk_body(t, _):
            b = lax.rem(t, _NB)
            for bb in range(_NB):
                @pl.when(b == bb)
                def _():
                    wait(t, bb)

                    @pl.when(t + _NB - 1 < nchk)
                    def _():
                        start(t + _NB - 1, (bb + _NB - 1) % _NB)

                    def grp(i, carry):
                        a0, a1 = carry
                        for r in range(_UNR):
                            off = (i * _UNR + r) * _L
                            dv = db[bb, pl.ds(off, _L)]
                            sv = sb[bb, pl.ds(off, _L)]
                            x0 = plsc.load_gather(tats, [dv])
                            y0 = plsc.load_gather(ta, [sv])
                            a0 = a0 + x0 * y0
                            x1 = plsc.load_gather(tats, [dv + nv])
                            y1 = plsc.load_gather(ta, [sv + nv])
                            a1 = a1 + x1 * y1
                        return (a0, a1)

                    z = jnp.zeros((_L,), jnp.float32)
                    acc = lax.fori_loop(0, groups // _UNR, grp, (z, z))
                    accv[pl.ds(0, _L)] = accv[pl.ds(0, _L)] + acc[0]
                    accv[pl.ds(_L, _L)] = accv[pl.ds(_L, _L)] + acc[1]
            return 0

        lax.fori_loop(0, nchk, chunk_body, 0)
        pltpu.sync_copy(accv, out_hbm.at[tid])

    return edge_sum


def kernel(x, edge_index, affiliate_logits, community_scalars, feat_mat):
    n, c = x.shape
    k = affiliate_logits.shape[1]
    e = edge_index.shape[1]

    a_tab, ats_tab, gl, fl = _tc_call(
        affiliate_logits, x, community_scalars.reshape(1, k), feat_mat,
        blk=2000)

    a_cols = a_tab.T.reshape(_NW, 2 * n)
    ats_cols = ats_tab.T.reshape(_NW, 2 * n)

    partials = _sc_edge_kernel(n, k, e)(
        edge_index[0], edge_index[1], ats_cols, a_cols)
    local = 2.0 * jnp.sum(partials)

    loss = (gl[0, 0] - local + jnp.float32(e)) / jnp.float32(n)
    return loss + fl[0, 0] / jnp.float32(c)
